# baseline (speedup 1.0000x reference)
import jax
import jax.numpy as jnp
from jax import lax
from jax.experimental import pallas as pl
from jax.experimental.pallas import tpu as pltpu
from jax.experimental.pallas import tpu_sc as plsc

D = 128
B = 16384
NC = 2
NS = 16
L = 16
NW = NC * NS
BPW = B // NW
CB = 128
NCHUNK = BPW // CB
NG = CB // L


def _body(idx0_hbm, idx1_hbm, w_hbm, out_hbm,
          i0a, i1a, i0b, i1b, r0a, r1a, r0b, r1b, out_v,
          s0a, s1a, s0b, s1b):
    wid = lax.axis_index("s") * NC + lax.axis_index("c")
    base = wid * BPW
    bufs = ((i0a, i1a, r0a, r1a, s0a, s1a),
            (i0b, i1b, r0b, r1b, s0b, s1b))

    def issue(c, slot):
        i0, i1, r0, r1, s0, s1 = bufs[slot]
        cbase = base + c * CB
        pltpu.sync_copy(idx0_hbm.at[pl.ds(cbase, CB)], i0)
        pltpu.sync_copy(idx1_hbm.at[pl.ds(cbase, CB)], i1)
        pltpu.async_copy(w_hbm.at[i0], r0, s0)
        pltpu.async_copy(w_hbm.at[i1], r1, s1)

    def wait(slot):
        i0, i1, r0, r1, s0, s1 = bufs[slot]
        pltpu.make_async_copy(w_hbm.at[i0], r0, s0).wait()
        pltpu.make_async_copy(w_hbm.at[i1], r1, s1).wait()

    issue(0, 0)
    for c in range(NCHUNK):
        slot = c % 2
        if c + 1 < NCHUNK:
            issue(c + 1, 1 - slot)
        wait(slot)
        _, _, r0, r1, _, _ = bufs[slot]

        def group(g, carry, r0=r0, r1=r1, c=c):
            rid = lax.iota(jnp.int32, L) + g * L
            accs = [jnp.zeros((L,), jnp.float32) for _ in range(4)]
            for dd in range(D):
                col = jnp.full((L,), dd, jnp.int32)
                v0 = plsc.load_gather(r0, [rid, col])
                v1 = plsc.load_gather(r1, [rid, col])
                accs[dd % 4] = accs[dd % 4] + v0 * v1
            acc = (accs[0] + accs[1]) + (accs[2] + accs[3])
            out_v[pl.ds(c * CB + g * L, L)] = acc
            return carry

        lax.fori_loop(0, NG, group, 0)

    pltpu.sync_copy(out_v, out_hbm.at[pl.ds(base, BPW)])


def kernel(X_idxs, W):
    idx0 = X_idxs[:, 0].astype(jnp.int32)
    idx1 = X_idxs[:, 1].astype(jnp.int32)
    mesh = plsc.VectorSubcoreMesh(core_axis_name="c", subcore_axis_name="s")
    f = pl.kernel(
        _body,
        out_type=jax.ShapeDtypeStruct((B,), jnp.float32),
        mesh=mesh,
        compiler_params=pltpu.CompilerParams(needs_layout_passes=False),
        scratch_types=[
            pltpu.VMEM((CB,), jnp.int32),
            pltpu.VMEM((CB,), jnp.int32),
            pltpu.VMEM((CB,), jnp.int32),
            pltpu.VMEM((CB,), jnp.int32),
            pltpu.VMEM((CB, D), jnp.float32),
            pltpu.VMEM((CB, D), jnp.float32),
            pltpu.VMEM((CB, D), jnp.float32),
            pltpu.VMEM((CB, D), jnp.float32),
            pltpu.VMEM((BPW,), jnp.float32),
            pltpu.SemaphoreType.DMA,
            pltpu.SemaphoreType.DMA,
            pltpu.SemaphoreType.DMA,
            pltpu.SemaphoreType.DMA,
        ],
    )
    return f(idx0, idx1, W)


# speedup vs baseline: 2.1301x; 2.1301x over previous
import jax
import jax.numpy as jnp
from jax import lax
from jax.experimental import pallas as pl
from jax.experimental.pallas import tpu as pltpu
from jax.experimental.pallas import tpu_sc as plsc

D = 128
B = 16384
NC = 2
NS = 16
L = 16
NW = NC * NS
BPW = B // NW
CB = 128
NCHUNK = BPW // CB
NG = CB // L


def _body(idx0_hbm, idx1_hbm, w_hbm, out_hbm,
          i0a, i1a, i0b, i1b, r0a, r1a, r0b, r1b, out_v,
          s0a, s1a, s0b, s1b):
    wid = lax.axis_index("s") * NC + lax.axis_index("c")
    base = wid * BPW
    bufs = ((i0a, i1a, r0a, r1a, s0a, s1a),
            (i0b, i1b, r0b, r1b, s0b, s1b))

    def issue(c, slot):
        i0, i1, r0, r1, s0, s1 = bufs[slot]
        cbase = base + c * CB
        pltpu.sync_copy(idx0_hbm.at[pl.ds(cbase, CB)], i0)
        pltpu.sync_copy(idx1_hbm.at[pl.ds(cbase, CB)], i1)
        pltpu.async_copy(w_hbm.at[i0], r0, s0)
        pltpu.async_copy(w_hbm.at[i1], r1, s1)

    def wait(slot):
        i0, i1, r0, r1, s0, s1 = bufs[slot]
        pltpu.make_async_copy(w_hbm.at[i0], r0, s0).wait()
        pltpu.make_async_copy(w_hbm.at[i1], r1, s1).wait()

    issue(0, 0)
    for c in range(NCHUNK):
        slot = c % 2
        if c + 1 < NCHUNK:
            issue(c + 1, 1 - slot)
        wait(slot)
        _, _, r0, r1, _, _ = bufs[slot]

        lanes = lax.iota(jnp.int32, L)

        def group(g, carry, r0=r0, r1=r1, c=c):
            ebase = g * L
            vec = jnp.zeros((L,), jnp.float32)
            for l in range(L):
                e = ebase + l
                acc = r0[e, pl.ds(0, L)] * r1[e, pl.ds(0, L)]
                for s in range(1, D // L):
                    acc = acc + (r0[e, pl.ds(s * L, L)]
                                 * r1[e, pl.ds(s * L, L)])
                red = jnp.sum(acc)
                vec = jnp.where(lanes == l, red, vec)
            out_v[pl.ds(c * CB + g * L, L)] = vec
            return carry

        lax.fori_loop(0, NG, group, 0)

    pltpu.sync_copy(out_v, out_hbm.at[pl.ds(base, BPW)])


def kernel(X_idxs, W):
    idx0 = X_idxs[:, 0].astype(jnp.int32)
    idx1 = X_idxs[:, 1].astype(jnp.int32)
    mesh = plsc.VectorSubcoreMesh(core_axis_name="c", subcore_axis_name="s")
    f = pl.kernel(
        _body,
        out_type=jax.ShapeDtypeStruct((B,), jnp.float32),
        mesh=mesh,
        compiler_params=pltpu.CompilerParams(needs_layout_passes=False),
        scratch_types=[
            pltpu.VMEM((CB,), jnp.int32),
            pltpu.VMEM((CB,), jnp.int32),
            pltpu.VMEM((CB,), jnp.int32),
            pltpu.VMEM((CB,), jnp.int32),
            pltpu.VMEM((CB, D), jnp.float32),
            pltpu.VMEM((CB, D), jnp.float32),
            pltpu.VMEM((CB, D), jnp.float32),
            pltpu.VMEM((CB, D), jnp.float32),
            pltpu.VMEM((BPW,), jnp.float32),
            pltpu.SemaphoreType.DMA,
            pltpu.SemaphoreType.DMA,
            pltpu.SemaphoreType.DMA,
            pltpu.SemaphoreType.DMA,
        ],
    )
    return f(idx0, idx1, W)


# speedup vs baseline: 3.2233x; 1.5132x over previous
import jax
import jax.numpy as jnp
from jax import lax
from jax.experimental import pallas as pl
from jax.experimental.pallas import tpu as pltpu
from jax.experimental.pallas import tpu_sc as plsc

D = 128
B = 16384
NC = 2
NS = 16
L = 16
NW = NC * NS
BPW = B // NW
CB = 128
NCHUNK = BPW // CB
NG = CB // L


def _body(idx0_hbm, idx1_hbm, w_hbm, out_hbm,
          i0a, i1a, i0b, i1b, r0a, r1a, r0b, r1b, out_v,
          s0a, s1a, s0b, s1b):
    wid = lax.axis_index("s") * NC + lax.axis_index("c")
    base = wid * BPW
    bufs = ((i0a, i1a, r0a, r1a, s0a, s1a),
            (i0b, i1b, r0b, r1b, s0b, s1b))

    def issue(c, slot):
        i0, i1, r0, r1, s0, s1 = bufs[slot]
        cbase = base + c * CB
        pltpu.sync_copy(idx0_hbm.at[pl.ds(cbase, CB)], i0)
        pltpu.sync_copy(idx1_hbm.at[pl.ds(cbase, CB)], i1)
        pltpu.async_copy(w_hbm.at[i0], r0, s0)
        pltpu.async_copy(w_hbm.at[i1], r1, s1)

    def wait(slot):
        i0, i1, r0, r1, s0, s1 = bufs[slot]
        pltpu.make_async_copy(w_hbm.at[i0], r0, s0).wait()
        pltpu.make_async_copy(w_hbm.at[i1], r1, s1).wait()

    issue(0, 0)
    for c in range(NCHUNK):
        slot = c % 2
        if c + 1 < NCHUNK:
            issue(c + 1, 1 - slot)
        wait(slot)
        _, _, r0, r1, _, _ = bufs[slot]

        lanes = lax.iota(jnp.int32, L)
        last_lane = lanes == (L - 1)

        @plsc.parallel_loop(0, CB, 1, unroll=4)
        def _(e, r0=r0, r1=r1, c=c):
            acc = r0[e, pl.ds(0, L)] * r1[e, pl.ds(0, L)]
            for s in range(1, D // L):
                acc = acc + (r0[e, pl.ds(s * L, L)]
                             * r1[e, pl.ds(s * L, L)])
            scn = plsc.cumsum(acc)
            pos = jnp.full((L,), c * CB + e, jnp.int32)
            plsc.store_scatter(out_v, [pos], scn, mask=last_lane)

    pltpu.sync_copy(out_v, out_hbm.at[pl.ds(base, BPW)])


def kernel(X_idxs, W):
    idx0 = X_idxs[:, 0].astype(jnp.int32)
    idx1 = X_idxs[:, 1].astype(jnp.int32)
    mesh = plsc.VectorSubcoreMesh(core_axis_name="c", subcore_axis_name="s")
    f = pl.kernel(
        _body,
        out_type=jax.ShapeDtypeStruct((B,), jnp.float32),
        mesh=mesh,
        compiler_params=pltpu.CompilerParams(needs_layout_passes=False),
        scratch_types=[
            pltpu.VMEM((CB,), jnp.int32),
            pltpu.VMEM((CB,), jnp.int32),
            pltpu.VMEM((CB,), jnp.int32),
            pltpu.VMEM((CB,), jnp.int32),
            pltpu.VMEM((CB, D), jnp.float32),
            pltpu.VMEM((CB, D), jnp.float32),
            pltpu.VMEM((CB, D), jnp.float32),
            pltpu.VMEM((CB, D), jnp.float32),
            pltpu.VMEM((BPW,), jnp.float32),
            pltpu.SemaphoreType.DMA,
            pltpu.SemaphoreType.DMA,
            pltpu.SemaphoreType.DMA,
            pltpu.SemaphoreType.DMA,
        ],
    )
    return f(idx0, idx1, W)


# speedup vs baseline: 3.5743x; 1.1089x over previous
import jax
import jax.numpy as jnp
from jax import lax
from jax.experimental import pallas as pl
from jax.experimental.pallas import tpu as pltpu
from jax.experimental.pallas import tpu_sc as plsc

D = 128
B = 16384
NC = 2
NS = 16
L = 16
NW = NC * NS
BPW = B // NW
CB = 128
NCHUNK = BPW // CB
NG = CB // L


def _body(idx0_hbm, idx1_hbm, w_hbm, out_hbm,
          i0a, i1a, i0b, i1b, r0a, r1a, r0b, r1b, out_v,
          s0a, s1a, s0b, s1b):
    wid = lax.axis_index("s") * NC + lax.axis_index("c")
    base = wid * BPW
    bufs = ((i0a, i1a, r0a, r1a, s0a, s1a),
            (i0b, i1b, r0b, r1b, s0b, s1b))

    def issue(c, slot):
        i0, i1, r0, r1, s0, s1 = bufs[slot]
        cbase = base + c * CB
        pltpu.sync_copy(idx0_hbm.at[pl.ds(cbase, CB)], i0)
        pltpu.sync_copy(idx1_hbm.at[pl.ds(cbase, CB)], i1)
        pltpu.async_copy(w_hbm.at[i0], r0, s0)
        pltpu.async_copy(w_hbm.at[i1], r1, s1)

    def wait(slot):
        i0, i1, r0, r1, s0, s1 = bufs[slot]
        pltpu.make_async_copy(w_hbm.at[i0], r0, s0).wait()
        pltpu.make_async_copy(w_hbm.at[i1], r1, s1).wait()

    issue(0, 0)
    for c in range(NCHUNK):
        slot = c % 2
        if c + 1 < NCHUNK:
            issue(c + 1, 1 - slot)
        wait(slot)
        _, _, r0, r1, _, _ = bufs[slot]

        lanes = lax.iota(jnp.int32, L)
        last_lane = lanes == (L - 1)

        @plsc.parallel_loop(0, CB, 1, unroll=4)
        def _(e, r0=r0, r1=r1, c=c):
            acc0 = jnp.zeros((L,), jnp.float32)
            acc1 = jnp.zeros((L,), jnp.float32)
            for s in range(D // (2 * L)):
                x0 = r0[e, pl.ds(s * 2 * L, 2 * L)]
                x1 = r1[e, pl.ds(s * 2 * L, 2 * L)]
                a0, b0 = plsc.unpack(x0, format=plsc.PackFormat.INTERLEAVED)
                a1, b1 = plsc.unpack(x1, format=plsc.PackFormat.INTERLEAVED)
                acc0 = acc0 + a0 * a1
                acc1 = acc1 + b0 * b1
            scn = plsc.cumsum(acc0 + acc1)
            pos = jnp.full((L,), c * CB + e, jnp.int32)
            plsc.store_scatter(out_v, [pos], scn, mask=last_lane)

    pltpu.sync_copy(out_v, out_hbm.at[pl.ds(base, BPW)])


def kernel(X_idxs, W):
    idx0 = X_idxs[:, 0].astype(jnp.int32)
    idx1 = X_idxs[:, 1].astype(jnp.int32)
    w_bf = W.astype(jnp.bfloat16)
    mesh = plsc.VectorSubcoreMesh(core_axis_name="c", subcore_axis_name="s")
    f = pl.kernel(
        _body,
        out_type=jax.ShapeDtypeStruct((B,), jnp.float32),
        mesh=mesh,
        compiler_params=pltpu.CompilerParams(
            needs_layout_passes=False, use_tc_tiling_on_sc=False),
        scratch_types=[
            pltpu.VMEM((CB,), jnp.int32),
            pltpu.VMEM((CB,), jnp.int32),
            pltpu.VMEM((CB,), jnp.int32),
            pltpu.VMEM((CB,), jnp.int32),
            pltpu.VMEM((CB, D), jnp.bfloat16),
            pltpu.VMEM((CB, D), jnp.bfloat16),
            pltpu.VMEM((CB, D), jnp.bfloat16),
            pltpu.VMEM((CB, D), jnp.bfloat16),
            pltpu.VMEM((BPW,), jnp.float32),
            pltpu.SemaphoreType.DMA,
            pltpu.SemaphoreType.DMA,
            pltpu.SemaphoreType.DMA,
            pltpu.SemaphoreType.DMA,
        ],
    )
    return f(idx0, idx1, w_bf)
